# Initial kernel scaffold; baseline (speedup 1.0000x reference)
import jax
import jax.numpy as jnp
from jax.experimental import pallas as pl


def kernel(x, edge_index, W1, b1, W2, b2):
    raise NotImplementedError("write your pallas kernel here")


# speedup vs baseline: 39.4471x; 39.4471x over previous
import functools

import jax
import jax.numpy as jnp
from jax import lax
from jax.experimental import pallas as pl
from jax.experimental.pallas import tpu as pltpu
from jax.experimental.pallas import tpu_sc as plsc

_N = 10000
_E = 320000
_D = 128
_F = 16
_C = 10

_NP = 10240
_ROWS_PER_TILE = _NP // 16

_NW = 32
_K = 128
_CHUNKS = 80
_EP = _NW * _CHUNKS * _K

_TC_BLOCK = 1024
_GRID = _NP // _TC_BLOCK

_sc_mesh = functools.partial(
    plsc.VectorSubcoreMesh, core_axis_name="c", subcore_axis_name="s"
)


def _deg_body(src_hbm, out_hbm, idx_v, hist):
    cid = lax.axis_index("c")
    sid = lax.axis_index("s")
    wid = sid * 2 + cid
    pltpu.sync_copy(src_hbm.at[wid], idx_v)

    zeros = jnp.zeros((16,), jnp.float32)

    def zero_body(i, _):
        hist[pl.ds(i * 16, 16)] = zeros
        return 0

    lax.fori_loop(0, _NP // 16, zero_body, 0)

    ones = jnp.ones((16,), jnp.float32)

    def body(c, _):
        for j in range(_K // 16):
            idx = idx_v[c, pl.ds(j * 16, 16)]
            plsc.addupdate_scatter(hist, [idx], ones)
        return 0

    lax.fori_loop(0, _CHUNKS, body, 0)
    pltpu.sync_copy(hist, out_hbm.at[wid])


@functools.partial(
    pl.kernel,
    out_type=jax.ShapeDtypeStruct((_NW, _NP), jnp.float32),
    mesh=_sc_mesh(),
    scratch_types=[
        pltpu.VMEM((_CHUNKS, _K), jnp.int32),
        pltpu.VMEM((_NP,), jnp.float32),
    ],
    compiler_params=pltpu.CompilerParams(needs_layout_passes=False),
)
def _sc_degree(src_hbm, out_hbm, idx_v, hist):
    _deg_body(src_hbm, out_hbm, idx_v, hist)


def _scat_body(g_hbm, src_hbm, dst_hbm, out_hbm,
               sidx, didx, rows0, rows1, zbuf, acc, gsem):
    cid = lax.axis_index("c")
    sid = lax.axis_index("s")
    wid = sid * 2 + cid
    pltpu.sync_copy(src_hbm.at[wid], sidx)
    pltpu.sync_copy(dst_hbm.at[wid], didx)

    zeros = jnp.zeros((16,), jnp.float32)

    def zero_body(i, _):
        zbuf[i, :] = zeros
        return 0

    lax.fori_loop(0, _ROWS_PER_TILE, zero_body, 0)
    base = sid * _ROWS_PER_TILE
    pltpu.sync_copy(zbuf, acc.at[pl.ds(base, _ROWS_PER_TILE)])
    plsc.subcore_barrier()

    rows = (rows0, rows1)

    pltpu.make_async_copy(g_hbm.at[sidx.at[0]], rows0, gsem).start()

    def body(c, _):
        for b in range(2):
            cc = c + b
            buf = rows[b]
            pltpu.make_async_copy(g_hbm.at[sidx.at[cc]], buf, gsem).wait()

            @pl.when(cc + 1 < _CHUNKS)
            def _start_next():
                pltpu.make_async_copy(
                    g_hbm.at[sidx.at[cc + 1]], rows[1 - b], gsem
                ).start()

            pltpu.sync_copy(buf, acc.at[didx.at[cc]], add=True)
        return 0

    lax.fori_loop(0, _CHUNKS // 2, lambda i, s: body(i * 2, s), 0)

    plsc.subcore_barrier()
    pltpu.sync_copy(
        acc.at[pl.ds(base, _ROWS_PER_TILE)],
        out_hbm.at[cid, pl.ds(base, _ROWS_PER_TILE)],
    )


@functools.partial(
    pl.kernel,
    out_type=jax.ShapeDtypeStruct((2, _NP, _F), jnp.float32),
    mesh=_sc_mesh(),
    scratch_types=[
        pltpu.VMEM((_CHUNKS, _K), jnp.int32),
        pltpu.VMEM((_CHUNKS, _K), jnp.int32),
        pltpu.VMEM((_K, _F), jnp.float32),
        pltpu.VMEM((_K, _F), jnp.float32),
        pltpu.VMEM((_ROWS_PER_TILE, _F), jnp.float32),
        pltpu.VMEM_SHARED((_NP, _F), jnp.float32),
        pltpu.SemaphoreType.DMA,
    ],
    compiler_params=pltpu.CompilerParams(use_tc_tiling_on_sc=False),
)
def _sc_scatter(g_hbm, src_hbm, dst_hbm, out_hbm,
                sidx, didx, rows0, rows1, zbuf, acc, gsem):
    _scat_body(g_hbm, src_hbm, dst_hbm, out_hbm,
               sidx, didx, rows0, rows1, zbuf, acc, gsem)


def _dis_from_parts(degp):
    deg = jnp.sum(degp, axis=0) + 1.0
    return lax.rsqrt(deg)


def _tc_a_body(degp_ref, x_ref, w1_ref, b1_ref, g_ref):
    dis = _dis_from_parts(degp_ref[...])
    xw = lax.dot_general(
        x_ref[...], w1_ref[...], (((1,), (1,)), ((), ())),
        preferred_element_type=jnp.float32,
    )
    g_ref[...] = dis[:, None] * (xw + b1_ref[...])


def _tc_b_body(s_ref, g_ref, degp_ref, w2_ref, b2_ref, o_ref):
    dis = _dis_from_parts(degp_ref[...])
    h = dis[:, None] * (s_ref[0] + s_ref[1] + g_ref[...])
    h = jnp.maximum(h, 0.0)
    hw = lax.dot_general(
        h, w2_ref[...], (((1,), (1,)), ((), ())),
        preferred_element_type=jnp.float32,
    )
    o_ref[...] = dis[:, None] * (hw + b2_ref[...])


def _tc_c_body(s_ref, g_ref, degp_ref, o_ref):
    dis = _dis_from_parts(degp_ref[...])
    z = dis[:, None] * (s_ref[0] + s_ref[1] + g_ref[...])
    col = lax.broadcasted_iota(jnp.int32, z.shape, 1)
    valid = col < _C
    zm = jnp.where(valid, z, -jnp.inf)
    m = jnp.max(zm, axis=1, keepdims=True)
    e = jnp.where(valid, jnp.exp(z - m), 0.0)
    s = jnp.sum(e, axis=1, keepdims=True)
    o_ref[...] = z - m - jnp.log(s)


def _row_spec(shape_prefix=()):
    nd = len(shape_prefix)
    return pl.BlockSpec(
        shape_prefix + (_TC_BLOCK, _F),
        lambda i: (0,) * nd + (i, 0),
    )


_degp_spec = pl.BlockSpec((_NW, _TC_BLOCK), lambda i: (0, i))


def _tc_a(degp, x, W1, b1r):
    return pl.pallas_call(
        _tc_a_body,
        grid=(_GRID,),
        in_specs=[
            _degp_spec,
            pl.BlockSpec((_TC_BLOCK, _D), lambda i: (i, 0)),
            pl.BlockSpec((_F, _D), lambda i: (0, 0)),
            pl.BlockSpec((1, _F), lambda i: (0, 0)),
        ],
        out_specs=_row_spec(),
        out_shape=jax.ShapeDtypeStruct((_NP, _F), jnp.float32),
    )(degp, x, W1, b1r)


def _tc_b(s1, g1, degp, W2p, b2r):
    return pl.pallas_call(
        _tc_b_body,
        grid=(_GRID,),
        in_specs=[
            _row_spec((2,)),
            _row_spec(),
            _degp_spec,
            pl.BlockSpec((_F, _F), lambda i: (0, 0)),
            pl.BlockSpec((1, _F), lambda i: (0, 0)),
        ],
        out_specs=_row_spec(),
        out_shape=jax.ShapeDtypeStruct((_NP, _F), jnp.float32),
    )(s1, g1, degp, W2p, b2r)


def _tc_c(s2, g2, degp):
    return pl.pallas_call(
        _tc_c_body,
        grid=(_GRID,),
        in_specs=[_row_spec((2,)), _row_spec(), _degp_spec],
        out_specs=_row_spec(),
        out_shape=jax.ShapeDtypeStruct((_NP, _F), jnp.float32),
    )(s2, g2, degp)


def kernel(x, edge_index, W1, b1, W2, b2):
    src = edge_index[0]
    dst = edge_index[1]
    pad = _N + (jnp.arange(_EP - _E, dtype=jnp.int32) % (_NP - _N))
    src_p = jnp.concatenate([src, pad]).reshape(_NW, _CHUNKS, _K)
    dst_p = jnp.concatenate([dst, pad]).reshape(_NW, _CHUNKS, _K)

    x_p = jnp.pad(x, ((0, _NP - _N), (0, 0)))
    W2p = jnp.pad(W2, ((0, _F - _C), (0, 0)))
    b1r = b1.reshape(1, _F)
    b2r = jnp.pad(b2, (0, _F - _C)).reshape(1, _F)

    degp = _sc_degree(src_p)
    g1 = _tc_a(degp, x_p, W1, b1r)
    s1 = _sc_scatter(g1, src_p, dst_p)
    g2 = _tc_b(s1, g1, degp, W2p, b2r)
    s2 = _sc_scatter(g2, src_p, dst_p)
    out = _tc_c(s2, g2, degp)
    return out[:_N, :_C]


# speedup vs baseline: 40.1160x; 1.0170x over previous
import functools

import jax
import jax.numpy as jnp
from jax import lax
from jax.experimental import pallas as pl
from jax.experimental.pallas import tpu as pltpu
from jax.experimental.pallas import tpu_sc as plsc

_N = 10000
_E = 320000
_D = 128
_H = 16
_C = 10

_NP = 10240
_ROWS_PER_TILE = _NP // 16

_NW = 32
_K = 128
_CHUNKS = 80
_EP = _NW * _CHUNKS * _K

_TC_BLOCK = 1024
_GRID = _NP // _TC_BLOCK

_sc_mesh = functools.partial(
    plsc.VectorSubcoreMesh, core_axis_name="c", subcore_axis_name="s"
)


def _deg_body(src_hbm, out_hbm, idx_v, hist):
    cid = lax.axis_index("c")
    sid = lax.axis_index("s")
    wid = sid * 2 + cid
    pltpu.sync_copy(src_hbm.at[wid], idx_v)

    zeros = jnp.zeros((16,), jnp.float32)

    def zero_body(i, _):
        hist[pl.ds(i * 16, 16)] = zeros
        return 0

    lax.fori_loop(0, _NP // 16, zero_body, 0)

    ones = jnp.ones((16,), jnp.float32)

    def body(c, _):
        for j in range(_K // 16):
            idx = idx_v[c, pl.ds(j * 16, 16)]
            plsc.addupdate_scatter(hist, [idx], ones)
        return 0

    lax.fori_loop(0, _CHUNKS, body, 0)
    pltpu.sync_copy(hist, out_hbm.at[wid])


@functools.partial(
    pl.kernel,
    out_type=jax.ShapeDtypeStruct((_NW, _NP), jnp.float32),
    mesh=_sc_mesh(),
    scratch_types=[
        pltpu.VMEM((_CHUNKS, _K), jnp.int32),
        pltpu.VMEM((_NP,), jnp.float32),
    ],
    compiler_params=pltpu.CompilerParams(needs_layout_passes=False),
)
def _sc_degree(src_hbm, out_hbm, idx_v, hist):
    _deg_body(src_hbm, out_hbm, idx_v, hist)


def _scat_body(w, g_hbm, src_hbm, dst_hbm, zero_hbm, out_hbm,
               sidx, didx, rows0, rows1, acc, gsem):
    cid = lax.axis_index("c")
    sid = lax.axis_index("s")
    wid = sid * 2 + cid
    pltpu.sync_copy(src_hbm.at[wid], sidx)
    pltpu.sync_copy(dst_hbm.at[wid], didx)

    base = sid * _ROWS_PER_TILE
    pltpu.sync_copy(
        zero_hbm.at[pl.ds(base, _ROWS_PER_TILE)],
        acc.at[pl.ds(base, _ROWS_PER_TILE)],
    )
    plsc.subcore_barrier()

    rows = (rows0, rows1)

    pltpu.make_async_copy(g_hbm.at[sidx.at[0]], rows0, gsem).start()

    def body(c, _):
        for b in range(2):
            cc = c + b
            buf = rows[b]
            pltpu.make_async_copy(g_hbm.at[sidx.at[cc]], buf, gsem).wait()

            @pl.when(cc + 1 < _CHUNKS)
            def _start_next():
                pltpu.make_async_copy(
                    g_hbm.at[sidx.at[cc + 1]], rows[1 - b], gsem
                ).start()

            pltpu.sync_copy(buf, acc.at[didx.at[cc]], add=True)
        return 0

    lax.fori_loop(0, _CHUNKS // 2, lambda i, s: body(i * 2, s), 0)

    plsc.subcore_barrier()
    pltpu.sync_copy(
        acc.at[pl.ds(base, _ROWS_PER_TILE)],
        out_hbm.at[cid, pl.ds(base, _ROWS_PER_TILE)],
    )


def _make_sc_scatter(w):
    @functools.partial(
        pl.kernel,
        out_type=jax.ShapeDtypeStruct((2, _NP, w), jnp.float32),
        mesh=_sc_mesh(),
        scratch_types=[
            pltpu.VMEM((_CHUNKS, _K), jnp.int32),
            pltpu.VMEM((_CHUNKS, _K), jnp.int32),
            pltpu.VMEM((_K, w), jnp.float32),
            pltpu.VMEM((_K, w), jnp.float32),
            pltpu.VMEM_SHARED((_NP, w), jnp.float32),
            pltpu.SemaphoreType.DMA,
        ],
        compiler_params=pltpu.CompilerParams(use_tc_tiling_on_sc=False),
    )
    def _sc_scatter(g_hbm, src_hbm, dst_hbm, zero_hbm, out_hbm,
                    sidx, didx, rows0, rows1, acc, gsem):
        _scat_body(w, g_hbm, src_hbm, dst_hbm, zero_hbm, out_hbm,
                   sidx, didx, rows0, rows1, acc, gsem)

    return _sc_scatter


_sc_scatter_h = _make_sc_scatter(_H)


def _dis_from_parts(degp):
    deg = jnp.sum(degp, axis=0) + 1.0
    return lax.rsqrt(deg)


def _tc_a_body(degp_ref, x_ref, w1_ref, b1_ref, g_ref):
    dis = _dis_from_parts(degp_ref[...])
    xw = lax.dot_general(
        x_ref[...], w1_ref[...], (((1,), (1,)), ((), ())),
        preferred_element_type=jnp.float32,
    )
    g_ref[...] = dis[:, None] * (xw + b1_ref[...])


def _tc_b_body(s_ref, g_ref, degp_ref, w2_ref, b2_ref, o_ref):
    dis = _dis_from_parts(degp_ref[...])
    h = dis[:, None] * (s_ref[0] + s_ref[1] + g_ref[...])
    h = jnp.maximum(h, 0.0)
    hw = lax.dot_general(
        h, w2_ref[...], (((1,), (1,)), ((), ())),
        preferred_element_type=jnp.float32,
    )
    o_ref[...] = dis[:, None] * (hw + b2_ref[...])


def _tc_c_body(s_ref, g_ref, degp_ref, o_ref):
    dis = _dis_from_parts(degp_ref[...])
    z = dis[:, None] * (s_ref[0] + s_ref[1] + g_ref[...])
    col = lax.broadcasted_iota(jnp.int32, z.shape, 1)
    valid = col < _C
    zm = jnp.where(valid, z, -jnp.inf)
    m = jnp.max(zm, axis=1, keepdims=True)
    e = jnp.where(valid, jnp.exp(z - m), 0.0)
    s = jnp.sum(e, axis=1, keepdims=True)
    o_ref[...] = z - m - jnp.log(s)


def _row_spec(w, shape_prefix=()):
    nd = len(shape_prefix)
    return pl.BlockSpec(
        shape_prefix + (_TC_BLOCK, w),
        lambda i: (0,) * nd + (i, 0),
    )


_degp_spec = pl.BlockSpec((_NW, _TC_BLOCK), lambda i: (0, i))


def _tc_a(degp, x, W1, b1r):
    return pl.pallas_call(
        _tc_a_body,
        grid=(_GRID,),
        in_specs=[
            _degp_spec,
            pl.BlockSpec((_TC_BLOCK, _D), lambda i: (i, 0)),
            pl.BlockSpec((_H, _D), lambda i: (0, 0)),
            pl.BlockSpec((1, _H), lambda i: (0, 0)),
        ],
        out_specs=_row_spec(_H),
        out_shape=jax.ShapeDtypeStruct((_NP, _H), jnp.float32),
    )(degp, x, W1, b1r)


def _tc_b(s1, g1, degp, W2, b2r):
    return pl.pallas_call(
        _tc_b_body,
        grid=(_GRID,),
        in_specs=[
            _row_spec(_H, (2,)),
            _row_spec(_H),
            _degp_spec,
            pl.BlockSpec((_H, _H), lambda i: (0, 0)),
            pl.BlockSpec((1, _H), lambda i: (0, 0)),
        ],
        out_specs=_row_spec(_H),
        out_shape=jax.ShapeDtypeStruct((_NP, _H), jnp.float32),
    )(s1, g1, degp, W2, b2r)


def _tc_c(s2, g2, degp):
    return pl.pallas_call(
        _tc_c_body,
        grid=(_GRID,),
        in_specs=[_row_spec(_H, (2,)), _row_spec(_H), _degp_spec],
        out_specs=_row_spec(_H),
        out_shape=jax.ShapeDtypeStruct((_NP, _H), jnp.float32),
    )(s2, g2, degp)


def kernel(x, edge_index, W1, b1, W2, b2):
    src = edge_index[0]
    dst = edge_index[1]
    pad = _N + (jnp.arange(_EP - _E, dtype=jnp.int32) % (_NP - _N))
    src_p = jnp.concatenate([src, pad]).reshape(_NW, _CHUNKS, _K)
    dst_p = jnp.concatenate([dst, pad]).reshape(_NW, _CHUNKS, _K)

    x_p = jnp.pad(x, ((0, _NP - _N), (0, 0)))
    b1r = b1.reshape(1, _H)
    W2p = jnp.pad(W2, ((0, _H - _C), (0, 0)))
    b2r = jnp.pad(b2, (0, _H - _C)).reshape(1, _H)
    zero_h = jnp.zeros((_NP, _H), jnp.float32)

    degp = _sc_degree(src_p)
    g1 = _tc_a(degp, x_p, W1, b1r)
    s1 = _sc_scatter_h(g1, src_p, dst_p, zero_h)
    g2 = _tc_b(s1, g1, degp, W2p, b2r)
    s2 = _sc_scatter_h(g2, src_p, dst_p, zero_h)
    out = _tc_c(s2, g2, degp)
    return out[:_N, :_C]


# speedup vs baseline: 42.3715x; 1.0562x over previous
import functools

import jax
import jax.numpy as jnp
from jax import lax
from jax.experimental import pallas as pl
from jax.experimental.pallas import tpu as pltpu
from jax.experimental.pallas import tpu_sc as plsc

_N = 10000
_E = 320000
_D = 128
_H = 16
_C = 10

_NA = 10240
_ROWS_PER_TILE = _NA // 16

_NW = 32
_K = 128
_NROWS = _E // _K
_RB = _NROWS // _NW
_EXTRA_FROM = _NW - (_NROWS - _RB * _NW)
_RMAX = _RB + 1

_TC_BLOCK = 1024
_GRID = (_N + _TC_BLOCK - 1) // _TC_BLOCK

_sc_mesh = functools.partial(
    plsc.VectorSubcoreMesh, core_axis_name="c", subcore_axis_name="s"
)


def _worker_range(wid):
    start = wid * _RB + jnp.maximum(wid - _EXTRA_FROM, 0)
    nrows = _RB + jnp.where(wid >= _EXTRA_FROM, 1, 0)
    return start, nrows


def _deg_body(ei_hbm, out_hbm, idx_v, hist):
    cid = lax.axis_index("c")
    sid = lax.axis_index("s")
    wid = sid * 2 + cid
    start, nrows = _worker_range(wid)
    pltpu.sync_copy(ei_hbm.at[0, pl.ds(start, _RMAX)], idx_v)

    zeros = jnp.zeros((16,), jnp.float32)

    def zero_body(i, _):
        hist[pl.ds(i * 16, 16)] = zeros
        return 0

    lax.fori_loop(0, _N // 16, zero_body, 0)

    ones = jnp.ones((16,), jnp.float32)

    def body(c, _):
        for j in range(_K // 16):
            idx = idx_v[c, pl.ds(j * 16, 16)]
            plsc.addupdate_scatter(hist, [idx], ones)
        return 0

    lax.fori_loop(0, nrows, body, 0)
    pltpu.sync_copy(hist, out_hbm.at[wid])


@functools.partial(
    pl.kernel,
    out_type=jax.ShapeDtypeStruct((_NW, _N), jnp.float32),
    mesh=_sc_mesh(),
    scratch_types=[
        pltpu.VMEM((_RMAX, _K), jnp.int32),
        pltpu.VMEM((_N,), jnp.float32),
    ],
    compiler_params=pltpu.CompilerParams(
        needs_layout_passes=False, use_tc_tiling_on_sc=False
    ),
)
def _sc_degree(ei_hbm, out_hbm, idx_v, hist):
    _deg_body(ei_hbm, out_hbm, idx_v, hist)


def _scat_body(g_hbm, ei_hbm, zero_hbm, out_hbm,
               sidx, didx, rows0, rows1, acc, gsem):
    cid = lax.axis_index("c")
    sid = lax.axis_index("s")
    wid = sid * 2 + cid
    start, nrows = _worker_range(wid)
    pltpu.sync_copy(ei_hbm.at[0, pl.ds(start, _RMAX)], sidx)
    pltpu.sync_copy(ei_hbm.at[1, pl.ds(start, _RMAX)], didx)

    base = sid * _ROWS_PER_TILE
    pltpu.sync_copy(
        zero_hbm.at[pl.ds(base, _ROWS_PER_TILE)],
        acc.at[pl.ds(base, _ROWS_PER_TILE)],
    )
    plsc.subcore_barrier()

    rows = (rows0, rows1)

    pltpu.make_async_copy(g_hbm.at[sidx.at[0]], rows0, gsem).start()

    def pair(c, _):
        for b in range(2):
            cc = c * 2 + b
            buf = rows[b]
            pltpu.make_async_copy(g_hbm.at[sidx.at[cc]], buf, gsem).wait()

            @pl.when(cc + 1 < nrows)
            def _start_next():
                pltpu.make_async_copy(
                    g_hbm.at[sidx.at[cc + 1]], rows[1 - b], gsem
                ).start()

            pltpu.sync_copy(buf, acc.at[didx.at[cc]], add=True)
        return 0

    lax.fori_loop(0, _RB // 2, pair, 0)

    @pl.when(nrows > _RB)
    def _tail():
        pltpu.make_async_copy(g_hbm.at[sidx.at[_RB]], rows[_RB % 2], gsem).wait()
        pltpu.sync_copy(rows[_RB % 2], acc.at[didx.at[_RB]], add=True)

    plsc.subcore_barrier()
    pltpu.sync_copy(
        acc.at[pl.ds(base, _ROWS_PER_TILE)],
        out_hbm.at[cid, pl.ds(base, _ROWS_PER_TILE)],
    )


@functools.partial(
    pl.kernel,
    out_type=jax.ShapeDtypeStruct((2, _NA, _H), jnp.float32),
    mesh=_sc_mesh(),
    scratch_types=[
        pltpu.VMEM((_RMAX, _K), jnp.int32),
        pltpu.VMEM((_RMAX, _K), jnp.int32),
        pltpu.VMEM((_K, _H), jnp.float32),
        pltpu.VMEM((_K, _H), jnp.float32),
        pltpu.VMEM_SHARED((_NA, _H), jnp.float32),
        pltpu.SemaphoreType.DMA,
    ],
    compiler_params=pltpu.CompilerParams(use_tc_tiling_on_sc=False),
)
def _sc_scatter(g_hbm, ei_hbm, zero_hbm, out_hbm,
                sidx, didx, rows0, rows1, acc, gsem):
    _scat_body(g_hbm, ei_hbm, zero_hbm, out_hbm,
               sidx, didx, rows0, rows1, acc, gsem)


def _dis_from_parts(degp):
    deg = jnp.sum(degp, axis=0) + 1.0
    return lax.rsqrt(deg)


def _tc_a_body(degp_ref, x_ref, w1_ref, b1_ref, g_ref):
    dis = _dis_from_parts(degp_ref[...])
    xw = lax.dot_general(
        x_ref[...], w1_ref[...], (((1,), (1,)), ((), ())),
        preferred_element_type=jnp.float32,
    )
    g_ref[...] = dis[:, None] * (xw + b1_ref[...])


def _tc_b_body(s_ref, g_ref, degp_ref, w2_ref, b2_ref, o_ref):
    dis = _dis_from_parts(degp_ref[...])
    h = dis[:, None] * (s_ref[0] + s_ref[1] + g_ref[...])
    h = jnp.maximum(h, 0.0)
    hw = lax.dot_general(
        h, w2_ref[...], (((1,), (1,)), ((), ())),
        preferred_element_type=jnp.float32,
    )
    o_ref[...] = dis[:, None] * (hw + b2_ref[...])


def _tc_c_body(s_ref, g_ref, degp_ref, o_ref):
    dis = _dis_from_parts(degp_ref[...])
    z = dis[:, None] * (s_ref[0] + s_ref[1] + g_ref[...])
    col = lax.broadcasted_iota(jnp.int32, z.shape, 1)
    valid = col < _C
    zm = jnp.where(valid, z, -jnp.inf)
    m = jnp.max(zm, axis=1, keepdims=True)
    e = jnp.where(valid, jnp.exp(z - m), 0.0)
    s = jnp.sum(e, axis=1, keepdims=True)
    o_ref[...] = z - m - jnp.log(s)


def _row_spec(shape_prefix=()):
    nd = len(shape_prefix)
    return pl.BlockSpec(
        shape_prefix + (_TC_BLOCK, _H),
        lambda i: (0,) * nd + (i, 0),
    )


_degp_spec = pl.BlockSpec((_NW, _TC_BLOCK), lambda i: (0, i))


def _tc_a(degp, x, W1, b1r):
    return pl.pallas_call(
        _tc_a_body,
        grid=(_GRID,),
        in_specs=[
            _degp_spec,
            pl.BlockSpec((_TC_BLOCK, _D), lambda i: (i, 0)),
            pl.BlockSpec((_H, _D), lambda i: (0, 0)),
            pl.BlockSpec((1, _H), lambda i: (0, 0)),
        ],
        out_specs=_row_spec(),
        out_shape=jax.ShapeDtypeStruct((_N, _H), jnp.float32),
    )(degp, x, W1, b1r)


def _tc_b(s1, g1, degp, W2p, b2r):
    return pl.pallas_call(
        _tc_b_body,
        grid=(_GRID,),
        in_specs=[
            _row_spec((2,)),
            _row_spec(),
            _degp_spec,
            pl.BlockSpec((_H, _H), lambda i: (0, 0)),
            pl.BlockSpec((1, _H), lambda i: (0, 0)),
        ],
        out_specs=_row_spec(),
        out_shape=jax.ShapeDtypeStruct((_N, _H), jnp.float32),
    )(s1, g1, degp, W2p, b2r)


def _tc_c(s2, g2, degp):
    return pl.pallas_call(
        _tc_c_body,
        grid=(_GRID,),
        in_specs=[_row_spec((2,)), _row_spec(), _degp_spec],
        out_specs=_row_spec(),
        out_shape=jax.ShapeDtypeStruct((_N, _H), jnp.float32),
    )(s2, g2, degp)


def kernel(x, edge_index, W1, b1, W2, b2):
    ei3 = edge_index.reshape(2, _NROWS, _K)

    b1r = b1.reshape(1, _H)
    W2p = jnp.pad(W2, ((0, _H - _C), (0, 0)))
    b2r = jnp.pad(b2, (0, _H - _C)).reshape(1, _H)
    zero_h = jnp.zeros((_NA, _H), jnp.float32)

    degp = _sc_degree(ei3)
    g1 = _tc_a(degp, x, W1, b1r)
    s1 = _sc_scatter(g1, ei3, zero_h)
    g2 = _tc_b(s1, g1, degp, W2p, b2r)
    s2 = _sc_scatter(g2, ei3, zero_h)
    out = _tc_c(s2, g2, degp)
    return out[:, :_C]


# speedup vs baseline: 42.5259x; 1.0036x over previous
import functools

import jax
import jax.numpy as jnp
from jax import lax
from jax.experimental import pallas as pl
from jax.experimental.pallas import tpu as pltpu
from jax.experimental.pallas import tpu_sc as plsc

_N = 10000
_E = 320000
_D = 128
_H = 16
_C = 10

_NA = 10240
_ROWS_PER_TILE = _NA // 16

_NW = 32
_K = 128
_NROWS = _E // _K
_RB = _NROWS // _NW
_EXTRA_FROM = _NW - (_NROWS - _RB * _NW)
_RMAX = _RB + 1

_TC_BLOCK = 1024
_GRID = (_N + _TC_BLOCK - 1) // _TC_BLOCK

_sc_mesh = functools.partial(
    plsc.VectorSubcoreMesh, core_axis_name="c", subcore_axis_name="s"
)


def _worker_range(wid):
    start = wid * _RB + jnp.maximum(wid - _EXTRA_FROM, 0)
    nrows = _RB + jnp.where(wid >= _EXTRA_FROM, 1, 0)
    return start, nrows


def _deg_body(ei_hbm, out_hbm, idx_v, hist):
    cid = lax.axis_index("c")
    sid = lax.axis_index("s")
    wid = sid * 2 + cid
    start, nrows = _worker_range(wid)
    pltpu.sync_copy(ei_hbm.at[0, pl.ds(start, _RMAX)], idx_v)

    zeros = jnp.zeros((16,), jnp.float32)

    def zero_body(i, _):
        hist[pl.ds(i * 16, 16)] = zeros
        return 0

    lax.fori_loop(0, _N // 16, zero_body, 0)

    ones = jnp.ones((16,), jnp.float32)

    def body(c, _):
        for j in range(_K // 16):
            idx = idx_v[c, pl.ds(j * 16, 16)]
            plsc.addupdate_scatter(hist, [idx], ones)
        return 0

    lax.fori_loop(0, nrows, body, 0)
    pltpu.sync_copy(hist, out_hbm.at[wid])


@functools.partial(
    pl.kernel,
    out_type=jax.ShapeDtypeStruct((_NW, _N), jnp.float32),
    mesh=_sc_mesh(),
    scratch_types=[
        pltpu.VMEM((_RMAX, _K), jnp.int32),
        pltpu.VMEM((_N,), jnp.float32),
    ],
    compiler_params=pltpu.CompilerParams(
        needs_layout_passes=False, use_tc_tiling_on_sc=False
    ),
)
def _sc_degree(ei_hbm, out_hbm, idx_v, hist):
    _deg_body(ei_hbm, out_hbm, idx_v, hist)


def _scat_body(g_hbm, ei_hbm, zero_hbm, out_hbm,
               sidx, didx, rows0, rows1, acc, gsem):
    cid = lax.axis_index("c")
    sid = lax.axis_index("s")
    wid = sid * 2 + cid
    start, nrows = _worker_range(wid)
    pltpu.sync_copy(ei_hbm.at[0, pl.ds(start, _RMAX)], sidx)
    pltpu.sync_copy(ei_hbm.at[1, pl.ds(start, _RMAX)], didx)

    base = sid * _ROWS_PER_TILE

    @pl.when(cid == 0)
    def _init_g():
        pltpu.sync_copy(
            g_hbm.at[pl.ds(base, _ROWS_PER_TILE)],
            acc.at[pl.ds(base, _ROWS_PER_TILE)],
        )

    @pl.when(cid == 1)
    def _init_zero():
        pltpu.sync_copy(
            zero_hbm.at[pl.ds(base, _ROWS_PER_TILE)],
            acc.at[pl.ds(base, _ROWS_PER_TILE)],
        )

    plsc.subcore_barrier()

    rows = (rows0, rows1)

    pltpu.make_async_copy(g_hbm.at[sidx.at[0]], rows0, gsem).start()

    def pair(c, _):
        for b in range(2):
            cc = c * 2 + b
            buf = rows[b]
            pltpu.make_async_copy(g_hbm.at[sidx.at[cc]], buf, gsem).wait()

            @pl.when(cc + 1 < nrows)
            def _start_next():
                pltpu.make_async_copy(
                    g_hbm.at[sidx.at[cc + 1]], rows[1 - b], gsem
                ).start()

            pltpu.sync_copy(buf, acc.at[didx.at[cc]], add=True)
        return 0

    lax.fori_loop(0, _RB // 2, pair, 0)

    @pl.when(nrows > _RB)
    def _tail():
        pltpu.make_async_copy(g_hbm.at[sidx.at[_RB]], rows[_RB % 2], gsem).wait()
        pltpu.sync_copy(rows[_RB % 2], acc.at[didx.at[_RB]], add=True)

    plsc.subcore_barrier()
    pltpu.sync_copy(
        acc.at[pl.ds(base, _ROWS_PER_TILE)],
        out_hbm.at[cid, pl.ds(base, _ROWS_PER_TILE)],
    )


@functools.partial(
    pl.kernel,
    out_type=jax.ShapeDtypeStruct((2, _NA, _H), jnp.float32),
    mesh=_sc_mesh(),
    scratch_types=[
        pltpu.VMEM((_RMAX, _K), jnp.int32),
        pltpu.VMEM((_RMAX, _K), jnp.int32),
        pltpu.VMEM((_K, _H), jnp.float32),
        pltpu.VMEM((_K, _H), jnp.float32),
        pltpu.VMEM_SHARED((_NA, _H), jnp.float32),
        pltpu.SemaphoreType.DMA,
    ],
    compiler_params=pltpu.CompilerParams(use_tc_tiling_on_sc=False),
)
def _sc_scatter(g_hbm, ei_hbm, zero_hbm, out_hbm,
                sidx, didx, rows0, rows1, acc, gsem):
    _scat_body(g_hbm, ei_hbm, zero_hbm, out_hbm,
               sidx, didx, rows0, rows1, acc, gsem)


def _dis_from_parts(degp):
    deg = jnp.sum(degp, axis=0) + 1.0
    return lax.rsqrt(deg)


def _tc_mm1_body(x_ref, w1_ref, b1_ref, o_ref):
    xw = lax.dot_general(
        x_ref[...], w1_ref[...], (((1,), (1,)), ((), ())),
        preferred_element_type=jnp.float32,
    )
    o_ref[...] = xw + b1_ref[...]


def _tc_scale_body(degp_ref, xw_ref, g_ref):
    dis = _dis_from_parts(degp_ref[...])
    g_ref[...] = dis[:, None] * xw_ref[...]


def _tc_b_body(s_ref, degp_ref, w2_ref, b2_ref, o_ref):
    dis = _dis_from_parts(degp_ref[...])
    h = dis[:, None] * (s_ref[0] + s_ref[1])
    h = jnp.maximum(h, 0.0)
    hw = lax.dot_general(
        h, w2_ref[...], (((1,), (1,)), ((), ())),
        preferred_element_type=jnp.float32,
    )
    o_ref[...] = dis[:, None] * (hw + b2_ref[...])


def _tc_c_body(s_ref, degp_ref, o_ref):
    dis = _dis_from_parts(degp_ref[...])
    z = dis[:, None] * (s_ref[0] + s_ref[1])
    col = lax.broadcasted_iota(jnp.int32, z.shape, 1)
    valid = col < _C
    zm = jnp.where(valid, z, -jnp.inf)
    m = jnp.max(zm, axis=1, keepdims=True)
    e = jnp.where(valid, jnp.exp(z - m), 0.0)
    s = jnp.sum(e, axis=1, keepdims=True)
    o_ref[...] = z - m - jnp.log(s)


def _row_spec(shape_prefix=()):
    nd = len(shape_prefix)
    return pl.BlockSpec(
        shape_prefix + (_TC_BLOCK, _H),
        lambda i: (0,) * nd + (i, 0),
    )


_degp_spec = pl.BlockSpec((_NW, _TC_BLOCK), lambda i: (0, i))


def _tc_mm1(x, W1, b1r):
    return pl.pallas_call(
        _tc_mm1_body,
        grid=(_GRID,),
        in_specs=[
            pl.BlockSpec((_TC_BLOCK, _D), lambda i: (i, 0)),
            pl.BlockSpec((_H, _D), lambda i: (0, 0)),
            pl.BlockSpec((1, _H), lambda i: (0, 0)),
        ],
        out_specs=_row_spec(),
        out_shape=jax.ShapeDtypeStruct((_NA, _H), jnp.float32),
    )(x, W1, b1r)


def _tc_scale(degp, xw):
    return pl.pallas_call(
        _tc_scale_body,
        grid=(_GRID,),
        in_specs=[_degp_spec, _row_spec()],
        out_specs=_row_spec(),
        out_shape=jax.ShapeDtypeStruct((_NA, _H), jnp.float32),
    )(degp, xw)


def _tc_b(s1, degp, W2p, b2r):
    return pl.pallas_call(
        _tc_b_body,
        grid=(_GRID,),
        in_specs=[
            _row_spec((2,)),
            _degp_spec,
            pl.BlockSpec((_H, _H), lambda i: (0, 0)),
            pl.BlockSpec((1, _H), lambda i: (0, 0)),
        ],
        out_specs=_row_spec(),
        out_shape=jax.ShapeDtypeStruct((_NA, _H), jnp.float32),
    )(s1, degp, W2p, b2r)


def _tc_c(s2, degp):
    return pl.pallas_call(
        _tc_c_body,
        grid=(_GRID,),
        in_specs=[_row_spec((2,)), _degp_spec],
        out_specs=_row_spec(),
        out_shape=jax.ShapeDtypeStruct((_N, _H), jnp.float32),
    )(s2, degp)


def kernel(x, edge_index, W1, b1, W2, b2):
    ei3 = edge_index.reshape(2, _NROWS, _K)

    b1r = b1.reshape(1, _H)
    W2p = jnp.pad(W2, ((0, _H - _C), (0, 0)))
    b2r = jnp.pad(b2, (0, _H - _C)).reshape(1, _H)
    zero_h = jnp.zeros((_NA, _H), jnp.float32)

    degp = _sc_degree(ei3)
    xw1 = _tc_mm1(x, W1, b1r)
    g1 = _tc_scale(degp, xw1)
    s1 = _sc_scatter(g1, ei3, zero_h)
    g2 = _tc_b(s1, degp, W2p, b2r)
    s2 = _sc_scatter(g2, ei3, zero_h)
    out = _tc_c(s2, degp)
    return out[:, :_C]


# speedup vs baseline: 56.0344x; 1.3177x over previous
import functools

import jax
import jax.numpy as jnp
from jax import lax
from jax.experimental import pallas as pl
from jax.experimental.pallas import tpu as pltpu
from jax.experimental.pallas import tpu_sc as plsc

_N = 10000
_E = 320000
_D = 128
_H = 16
_C = 10

_NA = 10240
_ROWS_PER_TILE = _NA // 16

_NW = 32
_K = 128
_NROWS = _E // _K
_RB = _NROWS // _NW
_EXTRA_FROM = _NW - (_NROWS - _RB * _NW)
_RMAX = _RB + 1

_TC_BLOCK = 1024
_GRID = (_N + _TC_BLOCK - 1) // _TC_BLOCK

_sc_mesh = functools.partial(
    plsc.VectorSubcoreMesh, core_axis_name="c", subcore_axis_name="s"
)


def _worker_range(wid):
    start = wid * _RB + jnp.maximum(wid - _EXTRA_FROM, 0)
    nrows = _RB + jnp.where(wid >= _EXTRA_FROM, 1, 0)
    return start, nrows


def _deg_body(ei_hbm, out_hbm, idx_v, hist):
    cid = lax.axis_index("c")
    sid = lax.axis_index("s")
    wid = sid * 2 + cid
    start, nrows = _worker_range(wid)
    pltpu.sync_copy(ei_hbm.at[0, pl.ds(start, _RMAX)], idx_v)

    zeros = jnp.zeros((16,), jnp.float32)

    def zero_body(i, _):
        hist[pl.ds(i * 16, 16)] = zeros
        return 0

    lax.fori_loop(0, _N // 16, zero_body, 0)

    ones = jnp.ones((16,), jnp.float32)

    def body(c, _):
        for j in range(_K // 16):
            idx = idx_v[c, pl.ds(j * 16, 16)]
            plsc.addupdate_scatter(hist, [idx], ones)
        return 0

    lax.fori_loop(0, nrows, body, 0)
    pltpu.sync_copy(hist, out_hbm.at[wid])


@functools.partial(
    pl.kernel,
    out_type=jax.ShapeDtypeStruct((_NW, _N), jnp.float32),
    mesh=_sc_mesh(),
    scratch_types=[
        pltpu.VMEM((_RMAX, _K), jnp.int32),
        pltpu.VMEM((_N,), jnp.float32),
    ],
    compiler_params=pltpu.CompilerParams(
        needs_layout_passes=False, use_tc_tiling_on_sc=False
    ),
)
def _sc_degree(ei_hbm, out_hbm, idx_v, hist):
    _deg_body(ei_hbm, out_hbm, idx_v, hist)


def _scat_body(g_hbm, ei_hbm, zero_hbm, out_hbm,
               sidx, didx, rows0, rows1, rows2, rows3, acc, gsem, ssem):
    cid = lax.axis_index("c")
    sid = lax.axis_index("s")
    wid = sid * 2 + cid
    start, nrows = _worker_range(wid)
    pltpu.sync_copy(ei_hbm.at[0, pl.ds(start, _RMAX)], sidx)
    pltpu.sync_copy(ei_hbm.at[1, pl.ds(start, _RMAX)], didx)

    base = sid * _ROWS_PER_TILE

    @pl.when(cid == 0)
    def _init_g():
        pltpu.sync_copy(
            g_hbm.at[pl.ds(base, _ROWS_PER_TILE)],
            acc.at[pl.ds(base, _ROWS_PER_TILE)],
        )

    @pl.when(cid == 1)
    def _init_zero():
        pltpu.sync_copy(
            zero_hbm.at[pl.ds(base, _ROWS_PER_TILE)],
            acc.at[pl.ds(base, _ROWS_PER_TILE)],
        )

    plsc.subcore_barrier()

    rows = (rows0, rows1, rows2, rows3)

    def g_start(cc, slot):
        pltpu.make_async_copy(g_hbm.at[sidx.at[cc]], rows[slot], gsem).start()

    def g_wait(cc, slot):
        pltpu.make_async_copy(g_hbm.at[sidx.at[cc]], rows[slot], gsem).wait()

    def s_start(cc, slot):
        pltpu.async_copy(rows[slot], acc.at[didx.at[cc]], ssem, add=True)

    def s_wait(cc, slot):
        pltpu.make_async_copy(rows[slot], acc.at[didx.at[cc]], ssem).wait()

    g_start(0, 0)
    g_start(1, 1)

    def quad(q, _):
        for b in range(4):
            cc = q * 4 + b
            sl_next = (b + 2) % 4

            @pl.when(cc >= 2)
            def _drain():
                s_wait(cc - 2, sl_next)

            @pl.when(cc + 2 < nrows)
            def _prefetch():
                g_start(cc + 2, sl_next)

            g_wait(cc, b)
            s_start(cc, b)
        return 0

    lax.fori_loop(0, _RB // 4, quad, 0)

    for cc in (_RB - 2, _RB - 1):
        sl_next = (cc + 2) % 4
        s_wait(cc - 2, sl_next)

        @pl.when(cc + 2 < nrows)
        def _prefetch_t(cc=cc, sl_next=sl_next):
            g_start(cc + 2, sl_next)

        g_wait(cc, cc % 4)
        s_start(cc, cc % 4)

    @pl.when(nrows > _RB)
    def _extra():
        s_wait(_RB - 2, (_RB - 2) % 4)
        g_wait(_RB, _RB % 4)
        s_start(_RB, _RB % 4)

    @pl.when(nrows == _RB)
    def _drain_even():
        s_wait(_RB - 2, (_RB - 2) % 4)
        s_wait(_RB - 1, (_RB - 1) % 4)

    @pl.when(nrows > _RB)
    def _drain_odd():
        s_wait(_RB - 1, (_RB - 1) % 4)
        s_wait(_RB, _RB % 4)

    plsc.subcore_barrier()
    pltpu.sync_copy(
        acc.at[pl.ds(base, _ROWS_PER_TILE)],
        out_hbm.at[cid, pl.ds(base, _ROWS_PER_TILE)],
    )


@functools.partial(
    pl.kernel,
    out_type=jax.ShapeDtypeStruct((2, _NA, _H), jnp.float32),
    mesh=_sc_mesh(),
    scratch_types=[
        pltpu.VMEM((_RMAX, _K), jnp.int32),
        pltpu.VMEM((_RMAX, _K), jnp.int32),
        pltpu.VMEM((_K, _H), jnp.float32),
        pltpu.VMEM((_K, _H), jnp.float32),
        pltpu.VMEM((_K, _H), jnp.float32),
        pltpu.VMEM((_K, _H), jnp.float32),
        pltpu.VMEM_SHARED((_NA, _H), jnp.float32),
        pltpu.SemaphoreType.DMA,
        pltpu.SemaphoreType.DMA,
    ],
    compiler_params=pltpu.CompilerParams(use_tc_tiling_on_sc=False),
)
def _sc_scatter(g_hbm, ei_hbm, zero_hbm, out_hbm,
                sidx, didx, rows0, rows1, rows2, rows3, acc, gsem, ssem):
    _scat_body(g_hbm, ei_hbm, zero_hbm, out_hbm,
               sidx, didx, rows0, rows1, rows2, rows3, acc, gsem, ssem)


def _dis_from_parts(degp):
    deg = jnp.sum(degp, axis=0) + 1.0
    return lax.rsqrt(deg)


def _tc_mm1_body(x_ref, w1_ref, b1_ref, o_ref):
    xw = lax.dot_general(
        x_ref[...], w1_ref[...], (((1,), (1,)), ((), ())),
        preferred_element_type=jnp.float32,
    )
    o_ref[...] = xw + b1_ref[...]


def _tc_scale_body(degp_ref, xw_ref, g_ref):
    dis = _dis_from_parts(degp_ref[...])
    g_ref[...] = dis[:, None] * xw_ref[...]


def _tc_b_body(s_ref, degp_ref, w2_ref, b2_ref, o_ref):
    dis = _dis_from_parts(degp_ref[...])
    h = dis[:, None] * (s_ref[0] + s_ref[1])
    h = jnp.maximum(h, 0.0)
    hw = lax.dot_general(
        h, w2_ref[...], (((1,), (1,)), ((), ())),
        preferred_element_type=jnp.float32,
    )
    o_ref[...] = dis[:, None] * (hw + b2_ref[...])


def _tc_c_body(s_ref, degp_ref, o_ref):
    dis = _dis_from_parts(degp_ref[...])
    z = dis[:, None] * (s_ref[0] + s_ref[1])
    col = lax.broadcasted_iota(jnp.int32, z.shape, 1)
    valid = col < _C
    zm = jnp.where(valid, z, -jnp.inf)
    m = jnp.max(zm, axis=1, keepdims=True)
    e = jnp.where(valid, jnp.exp(z - m), 0.0)
    s = jnp.sum(e, axis=1, keepdims=True)
    o_ref[...] = z - m - jnp.log(s)


def _row_spec(shape_prefix=()):
    nd = len(shape_prefix)
    return pl.BlockSpec(
        shape_prefix + (_TC_BLOCK, _H),
        lambda i: (0,) * nd + (i, 0),
    )


_degp_spec = pl.BlockSpec((_NW, _TC_BLOCK), lambda i: (0, i))


def _tc_mm1(x, W1, b1r):
    return pl.pallas_call(
        _tc_mm1_body,
        grid=(_GRID,),
        in_specs=[
            pl.BlockSpec((_TC_BLOCK, _D), lambda i: (i, 0)),
            pl.BlockSpec((_H, _D), lambda i: (0, 0)),
            pl.BlockSpec((1, _H), lambda i: (0, 0)),
        ],
        out_specs=_row_spec(),
        out_shape=jax.ShapeDtypeStruct((_NA, _H), jnp.float32),
    )(x, W1, b1r)


def _tc_scale(degp, xw):
    return pl.pallas_call(
        _tc_scale_body,
        grid=(_GRID,),
        in_specs=[_degp_spec, _row_spec()],
        out_specs=_row_spec(),
        out_shape=jax.ShapeDtypeStruct((_NA, _H), jnp.float32),
    )(degp, xw)


def _tc_b(s1, degp, W2p, b2r):
    return pl.pallas_call(
        _tc_b_body,
        grid=(_GRID,),
        in_specs=[
            _row_spec((2,)),
            _degp_spec,
            pl.BlockSpec((_H, _H), lambda i: (0, 0)),
            pl.BlockSpec((1, _H), lambda i: (0, 0)),
        ],
        out_specs=_row_spec(),
        out_shape=jax.ShapeDtypeStruct((_NA, _H), jnp.float32),
    )(s1, degp, W2p, b2r)


def _tc_c(s2, degp):
    return pl.pallas_call(
        _tc_c_body,
        grid=(_GRID,),
        in_specs=[_row_spec((2,)), _degp_spec],
        out_specs=_row_spec(),
        out_shape=jax.ShapeDtypeStruct((_N, _H), jnp.float32),
    )(s2, degp)


def kernel(x, edge_index, W1, b1, W2, b2):
    ei3 = edge_index.reshape(2, _NROWS, _K)

    b1r = b1.reshape(1, _H)
    W2p = jnp.pad(W2, ((0, _H - _C), (0, 0)))
    b2r = jnp.pad(b2, (0, _H - _C)).reshape(1, _H)
    zero_h = jnp.zeros((_NA, _H), jnp.float32)

    degp = _sc_degree(ei3)
    xw1 = _tc_mm1(x, W1, b1r)
    g1 = _tc_scale(degp, xw1)
    s1 = _sc_scatter(g1, ei3, zero_h)
    g2 = _tc_b(s1, degp, W2p, b2r)
    s2 = _sc_scatter(g2, ei3, zero_h)
    out = _tc_c(s2, degp)
    return out[:, :_C]


# speedup vs baseline: 62.7900x; 1.1206x over previous
import functools

import jax
import jax.numpy as jnp
from jax import lax
from jax.experimental import pallas as pl
from jax.experimental.pallas import tpu as pltpu
from jax.experimental.pallas import tpu_sc as plsc

_N = 10000
_E = 320000
_D = 128
_H = 16
_C = 10

_NA = 10240
_ROWS_PER_TILE = _NA // 16

_NW = 32
_K = 128
_NROWS = _E // _K
_RB = _NROWS // _NW
_EXTRA_FROM = _NW - (_NROWS - _RB * _NW)
_RMAX = _RB + 1

_TC_BLOCK = 1024
_GRID = (_N + _TC_BLOCK - 1) // _TC_BLOCK

_sc_mesh = functools.partial(
    plsc.VectorSubcoreMesh, core_axis_name="c", subcore_axis_name="s"
)


def _worker_range(wid):
    start = wid * _RB + jnp.maximum(wid - _EXTRA_FROM, 0)
    nrows = _RB + jnp.where(wid >= _EXTRA_FROM, 1, 0)
    return start, nrows


def _deg_body(ei_hbm, out_hbm, idx_v, hist):
    cid = lax.axis_index("c")
    sid = lax.axis_index("s")
    wid = sid * 2 + cid
    start, nrows = _worker_range(wid)
    pltpu.sync_copy(ei_hbm.at[0, pl.ds(start, _RMAX)], idx_v)

    zeros = jnp.zeros((16,), jnp.float32)

    def zero_body(i, _):
        hist[pl.ds(i * 16, 16)] = zeros
        return 0

    lax.fori_loop(0, _N // 16, zero_body, 0)

    ones = jnp.ones((16,), jnp.float32)

    def body(c, _):
        for j in range(_K // 16):
            idx = idx_v[c, pl.ds(j * 16, 16)]
            plsc.addupdate_scatter(hist, [idx], ones)
        return 0

    lax.fori_loop(0, nrows, body, 0)
    pltpu.sync_copy(hist, out_hbm.at[wid])


@functools.partial(
    pl.kernel,
    out_type=jax.ShapeDtypeStruct((_NW, _N), jnp.float32),
    mesh=_sc_mesh(),
    scratch_types=[
        pltpu.VMEM((_RMAX, _K), jnp.int32),
        pltpu.VMEM((_N,), jnp.float32),
    ],
    compiler_params=pltpu.CompilerParams(
        needs_layout_passes=False, use_tc_tiling_on_sc=False
    ),
)
def _sc_degree(ei_hbm, out_hbm, idx_v, hist):
    _deg_body(ei_hbm, out_hbm, idx_v, hist)


def _scat_body(g_hbm, ei_hbm, zero_hbm, out_hbm,
               sidx, didx, rows0, rows1, rows2, rows3, rows4, rows5,
               rows6, rows7, acc, gsem, ssem):
    cid = lax.axis_index("c")
    sid = lax.axis_index("s")
    wid = sid * 2 + cid
    start, nrows = _worker_range(wid)
    pltpu.sync_copy(ei_hbm.at[0, pl.ds(start, _RMAX)], sidx)
    pltpu.sync_copy(ei_hbm.at[1, pl.ds(start, _RMAX)], didx)

    base = sid * _ROWS_PER_TILE

    @pl.when(cid == 0)
    def _init_g():
        pltpu.sync_copy(
            g_hbm.at[pl.ds(base, _ROWS_PER_TILE)],
            acc.at[pl.ds(base, _ROWS_PER_TILE)],
        )

    @pl.when(cid == 1)
    def _init_zero():
        pltpu.sync_copy(
            zero_hbm.at[pl.ds(base, _ROWS_PER_TILE)],
            acc.at[pl.ds(base, _ROWS_PER_TILE)],
        )

    plsc.subcore_barrier()

    rows = (rows0, rows1, rows2, rows3, rows4, rows5, rows6, rows7)
    _R = 8
    _A = 4

    def g_start(cc, slot):
        pltpu.make_async_copy(g_hbm.at[sidx.at[cc]], rows[slot], gsem).start()

    def g_wait(cc, slot):
        pltpu.make_async_copy(g_hbm.at[sidx.at[cc]], rows[slot], gsem).wait()

    def s_start(cc, slot):
        pltpu.async_copy(rows[slot], acc.at[didx.at[cc]], ssem, add=True)

    def s_wait(cc, slot):
        pltpu.make_async_copy(rows[slot], acc.at[didx.at[cc]], ssem).wait()

    for j in range(_A):
        g_start(j, j)

    _MAIN = (_RB - (_R - 2)) // _R * _R

    def step(cc, b):
        sl_next = (b + _A) % _R

        @pl.when(jnp.asarray(cc) >= _A)
        def _drain():
            s_wait(cc - _A, sl_next)

        @pl.when(jnp.asarray(cc) + _A < nrows)
        def _prefetch():
            g_start(cc + _A, sl_next)

        g_wait(cc, b)
        s_start(cc, b)

    def group(q, _):
        for b in range(_R):
            step(q * _R + b, b)
        return 0

    lax.fori_loop(0, _MAIN // _R, group, 0)

    for cc in range(_MAIN, _RB):
        step(cc, cc % _R)

    @pl.when(nrows > _RB)
    def _extra():
        s_wait(_RB - _A, (_RB - _A) % _R)
        g_wait(_RB, _RB % _R)
        s_start(_RB, _RB % _R)

    @pl.when(nrows == _RB)
    def _drain_even():
        for k in range(_RB - _A, _RB):
            s_wait(k, k % _R)

    @pl.when(nrows > _RB)
    def _drain_odd():
        for k in range(_RB - _A + 1, _RB + 1):
            s_wait(k, k % _R)

    plsc.subcore_barrier()
    pltpu.sync_copy(
        acc.at[pl.ds(base, _ROWS_PER_TILE)],
        out_hbm.at[cid, pl.ds(base, _ROWS_PER_TILE)],
    )


@functools.partial(
    pl.kernel,
    out_type=jax.ShapeDtypeStruct((2, _NA, _H), jnp.float32),
    mesh=_sc_mesh(),
    scratch_types=[
        pltpu.VMEM((_RMAX, _K), jnp.int32),
        pltpu.VMEM((_RMAX, _K), jnp.int32),
        pltpu.VMEM((_K, _H), jnp.float32),
        pltpu.VMEM((_K, _H), jnp.float32),
        pltpu.VMEM((_K, _H), jnp.float32),
        pltpu.VMEM((_K, _H), jnp.float32),
        pltpu.VMEM((_K, _H), jnp.float32),
        pltpu.VMEM((_K, _H), jnp.float32),
        pltpu.VMEM((_K, _H), jnp.float32),
        pltpu.VMEM((_K, _H), jnp.float32),
        pltpu.VMEM_SHARED((_NA, _H), jnp.float32),
        pltpu.SemaphoreType.DMA,
        pltpu.SemaphoreType.DMA,
    ],
    compiler_params=pltpu.CompilerParams(use_tc_tiling_on_sc=False),
)
def _sc_scatter(g_hbm, ei_hbm, zero_hbm, out_hbm,
                sidx, didx, rows0, rows1, rows2, rows3, rows4, rows5,
                rows6, rows7, acc, gsem, ssem):
    _scat_body(g_hbm, ei_hbm, zero_hbm, out_hbm,
               sidx, didx, rows0, rows1, rows2, rows3, rows4, rows5,
               rows6, rows7, acc, gsem, ssem)


def _dis_from_parts(degp):
    deg = jnp.sum(degp, axis=0) + 1.0
    return lax.rsqrt(deg)


def _tc_mm1_body(x_ref, w1_ref, b1_ref, o_ref):
    xw = lax.dot_general(
        x_ref[...], w1_ref[...], (((1,), (1,)), ((), ())),
        preferred_element_type=jnp.float32,
    )
    o_ref[...] = xw + b1_ref[...]


def _tc_scale_body(degp_ref, xw_ref, g_ref):
    dis = _dis_from_parts(degp_ref[...])
    g_ref[...] = dis[:, None] * xw_ref[...]


def _tc_b_body(s_ref, degp_ref, w2_ref, b2_ref, o_ref):
    dis = _dis_from_parts(degp_ref[...])
    h = dis[:, None] * (s_ref[0] + s_ref[1])
    h = jnp.maximum(h, 0.0)
    hw = lax.dot_general(
        h, w2_ref[...], (((1,), (1,)), ((), ())),
        preferred_element_type=jnp.float32,
    )
    o_ref[...] = dis[:, None] * (hw + b2_ref[...])


def _tc_c_body(s_ref, degp_ref, o_ref):
    dis = _dis_from_parts(degp_ref[...])
    z = dis[:, None] * (s_ref[0] + s_ref[1])
    col = lax.broadcasted_iota(jnp.int32, z.shape, 1)
    valid = col < _C
    zm = jnp.where(valid, z, -jnp.inf)
    m = jnp.max(zm, axis=1, keepdims=True)
    e = jnp.where(valid, jnp.exp(z - m), 0.0)
    s = jnp.sum(e, axis=1, keepdims=True)
    o_ref[...] = z - m - jnp.log(s)


def _row_spec(shape_prefix=()):
    nd = len(shape_prefix)
    return pl.BlockSpec(
        shape_prefix + (_TC_BLOCK, _H),
        lambda i: (0,) * nd + (i, 0),
    )


_degp_spec = pl.BlockSpec((_NW, _TC_BLOCK), lambda i: (0, i))


def _tc_mm1(x, W1, b1r):
    return pl.pallas_call(
        _tc_mm1_body,
        grid=(_GRID,),
        in_specs=[
            pl.BlockSpec((_TC_BLOCK, _D), lambda i: (i, 0)),
            pl.BlockSpec((_H, _D), lambda i: (0, 0)),
            pl.BlockSpec((1, _H), lambda i: (0, 0)),
        ],
        out_specs=_row_spec(),
        out_shape=jax.ShapeDtypeStruct((_NA, _H), jnp.float32),
    )(x, W1, b1r)


def _tc_scale(degp, xw):
    return pl.pallas_call(
        _tc_scale_body,
        grid=(_GRID,),
        in_specs=[_degp_spec, _row_spec()],
        out_specs=_row_spec(),
        out_shape=jax.ShapeDtypeStruct((_NA, _H), jnp.float32),
    )(degp, xw)


def _tc_b(s1, degp, W2p, b2r):
    return pl.pallas_call(
        _tc_b_body,
        grid=(_GRID,),
        in_specs=[
            _row_spec((2,)),
            _degp_spec,
            pl.BlockSpec((_H, _H), lambda i: (0, 0)),
            pl.BlockSpec((1, _H), lambda i: (0, 0)),
        ],
        out_specs=_row_spec(),
        out_shape=jax.ShapeDtypeStruct((_NA, _H), jnp.float32),
    )(s1, degp, W2p, b2r)


def _tc_c(s2, degp):
    return pl.pallas_call(
        _tc_c_body,
        grid=(_GRID,),
        in_specs=[_row_spec((2,)), _degp_spec],
        out_specs=_row_spec(),
        out_shape=jax.ShapeDtypeStruct((_N, _H), jnp.float32),
    )(s2, degp)


def kernel(x, edge_index, W1, b1, W2, b2):
    ei3 = edge_index.reshape(2, _NROWS, _K)

    b1r = b1.reshape(1, _H)
    W2p = jnp.pad(W2, ((0, _H - _C), (0, 0)))
    b2r = jnp.pad(b2, (0, _H - _C)).reshape(1, _H)
    zero_h = jnp.zeros((_NA, _H), jnp.float32)

    degp = _sc_degree(ei3)
    xw1 = _tc_mm1(x, W1, b1r)
    g1 = _tc_scale(degp, xw1)
    s1 = _sc_scatter(g1, ei3, zero_h)
    g2 = _tc_b(s1, degp, W2p, b2r)
    s2 = _sc_scatter(g2, ei3, zero_h)
    out = _tc_c(s2, degp)
    return out[:, :_C]


# speedup vs baseline: 69.2658x; 1.1031x over previous
import functools

import jax
import jax.numpy as jnp
from jax import lax
from jax.experimental import pallas as pl
from jax.experimental.pallas import tpu as pltpu
from jax.experimental.pallas import tpu_sc as plsc

_N = 10000
_E = 320000
_D = 128
_H = 16
_C = 10

_NA = 10240
_ROWS_PER_TILE = _NA // 16

_NW = 32
_K = 128
_NROWS = _E // _K
_RB = _NROWS // _NW
_EXTRA_FROM = _NW - (_NROWS - _RB * _NW)
_RMAX = _RB + 1

_TC_BLOCK = 1024
_GRID = (_N + _TC_BLOCK - 1) // _TC_BLOCK

_sc_mesh = functools.partial(
    plsc.VectorSubcoreMesh, core_axis_name="c", subcore_axis_name="s"
)


def _worker_range(wid):
    start = wid * _RB + jnp.maximum(wid - _EXTRA_FROM, 0)
    nrows = _RB + jnp.where(wid >= _EXTRA_FROM, 1, 0)
    return start, nrows


def _deg_body(ei_hbm, out_hbm, idx_v, hist):
    cid = lax.axis_index("c")
    sid = lax.axis_index("s")
    wid = sid * 2 + cid
    start, nrows = _worker_range(wid)
    pltpu.sync_copy(ei_hbm.at[0, pl.ds(start, _RMAX)], idx_v)

    zeros = jnp.zeros((16,), jnp.float32)

    def zero_body(i, _):
        hist[pl.ds(i * 16, 16)] = zeros
        return 0

    lax.fori_loop(0, _N // 16, zero_body, 0)

    ones = jnp.ones((16,), jnp.float32)

    def body(c, _):
        for j in range(_K // 16):
            idx = idx_v[c, pl.ds(j * 16, 16)]
            plsc.addupdate_scatter(hist, [idx], ones)
        return 0

    lax.fori_loop(0, nrows, body, 0)
    pltpu.sync_copy(hist, out_hbm.at[wid])


@functools.partial(
    pl.kernel,
    out_type=jax.ShapeDtypeStruct((_NW, _N), jnp.float32),
    mesh=_sc_mesh(),
    scratch_types=[
        pltpu.VMEM((_RMAX, _K), jnp.int32),
        pltpu.VMEM((_N,), jnp.float32),
    ],
    compiler_params=pltpu.CompilerParams(
        needs_layout_passes=False, use_tc_tiling_on_sc=False
    ),
)
def _sc_degree(ei_hbm, out_hbm, idx_v, hist):
    _deg_body(ei_hbm, out_hbm, idx_v, hist)


def _scat_body(g_hbm, ei_hbm, zero_hbm, out_hbm,
               sidx, didx, rows0, rows1, rows2, rows3, rows4, rows5,
               rows6, rows7, rows8, rows9, rows10, rows11, rows12, rows13,
               rows14, rows15, acc, gsem, ssem):
    cid = lax.axis_index("c")
    sid = lax.axis_index("s")
    wid = sid * 2 + cid
    start, nrows = _worker_range(wid)
    pltpu.sync_copy(ei_hbm.at[0, pl.ds(start, _RMAX)], sidx)
    pltpu.sync_copy(ei_hbm.at[1, pl.ds(start, _RMAX)], didx)

    base = sid * _ROWS_PER_TILE

    @pl.when(cid == 0)
    def _init_g():
        pltpu.sync_copy(
            g_hbm.at[pl.ds(base, _ROWS_PER_TILE)],
            acc.at[pl.ds(base, _ROWS_PER_TILE)],
        )

    @pl.when(cid == 1)
    def _init_zero():
        pltpu.sync_copy(
            zero_hbm.at[pl.ds(base, _ROWS_PER_TILE)],
            acc.at[pl.ds(base, _ROWS_PER_TILE)],
        )

    plsc.subcore_barrier()

    rows = (rows0, rows1, rows2, rows3, rows4, rows5, rows6, rows7,
            rows8, rows9, rows10, rows11, rows12, rows13, rows14, rows15)
    _R = 16
    _A = 8

    def g_start(cc, slot):
        pltpu.make_async_copy(g_hbm.at[sidx.at[cc]], rows[slot], gsem).start()

    def g_wait(cc, slot):
        pltpu.make_async_copy(g_hbm.at[sidx.at[cc]], rows[slot], gsem).wait()

    def s_start(cc, slot):
        pltpu.async_copy(rows[slot], acc.at[didx.at[cc]], ssem, add=True)

    def s_wait(cc, slot):
        pltpu.make_async_copy(rows[slot], acc.at[didx.at[cc]], ssem).wait()

    for j in range(_A):
        g_start(j, j)

    _MAIN = (_RB - (_R - 2)) // _R * _R

    def step(cc, b):
        sl_next = (b + _A) % _R

        @pl.when(jnp.asarray(cc) >= _A)
        def _drain():
            s_wait(cc - _A, sl_next)

        @pl.when(jnp.asarray(cc) + _A < nrows)
        def _prefetch():
            g_start(cc + _A, sl_next)

        g_wait(cc, b)
        s_start(cc, b)

    def group(q, _):
        for b in range(_R):
            step(q * _R + b, b)
        return 0

    lax.fori_loop(0, _MAIN // _R, group, 0)

    for cc in range(_MAIN, _RB):
        step(cc, cc % _R)

    @pl.when(nrows > _RB)
    def _extra():
        s_wait(_RB - _A, (_RB - _A) % _R)
        g_wait(_RB, _RB % _R)
        s_start(_RB, _RB % _R)

    @pl.when(nrows == _RB)
    def _drain_even():
        for k in range(_RB - _A, _RB):
            s_wait(k, k % _R)

    @pl.when(nrows > _RB)
    def _drain_odd():
        for k in range(_RB - _A + 1, _RB + 1):
            s_wait(k, k % _R)

    plsc.subcore_barrier()
    pltpu.sync_copy(
        acc.at[pl.ds(base, _ROWS_PER_TILE)],
        out_hbm.at[cid, pl.ds(base, _ROWS_PER_TILE)],
    )


@functools.partial(
    pl.kernel,
    out_type=jax.ShapeDtypeStruct((2, _NA, _H), jnp.float32),
    mesh=_sc_mesh(),
    scratch_types=[
        pltpu.VMEM((_RMAX, _K), jnp.int32),
        pltpu.VMEM((_RMAX, _K), jnp.int32),
        pltpu.VMEM((_K, _H), jnp.float32),
        pltpu.VMEM((_K, _H), jnp.float32),
        pltpu.VMEM((_K, _H), jnp.float32),
        pltpu.VMEM((_K, _H), jnp.float32),
        pltpu.VMEM((_K, _H), jnp.float32),
        pltpu.VMEM((_K, _H), jnp.float32),
        pltpu.VMEM((_K, _H), jnp.float32),
        pltpu.VMEM((_K, _H), jnp.float32),
        pltpu.VMEM((_K, _H), jnp.float32),
        pltpu.VMEM((_K, _H), jnp.float32),
        pltpu.VMEM((_K, _H), jnp.float32),
        pltpu.VMEM((_K, _H), jnp.float32),
        pltpu.VMEM((_K, _H), jnp.float32),
        pltpu.VMEM((_K, _H), jnp.float32),
        pltpu.VMEM((_K, _H), jnp.float32),
        pltpu.VMEM((_K, _H), jnp.float32),
        pltpu.VMEM_SHARED((_NA, _H), jnp.float32),
        pltpu.SemaphoreType.DMA,
        pltpu.SemaphoreType.DMA,
    ],
    compiler_params=pltpu.CompilerParams(use_tc_tiling_on_sc=False),
)
def _sc_scatter(g_hbm, ei_hbm, zero_hbm, out_hbm,
                sidx, didx, rows0, rows1, rows2, rows3, rows4, rows5,
                rows6, rows7, rows8, rows9, rows10, rows11, rows12, rows13,
                rows14, rows15, acc, gsem, ssem):
    _scat_body(g_hbm, ei_hbm, zero_hbm, out_hbm,
               sidx, didx, rows0, rows1, rows2, rows3, rows4, rows5,
               rows6, rows7, rows8, rows9, rows10, rows11, rows12, rows13,
               rows14, rows15, acc, gsem, ssem)


def _dis_from_parts(degp):
    deg = jnp.sum(degp, axis=0) + 1.0
    return lax.rsqrt(deg)


def _tc_mm1_body(x_ref, w1_ref, b1_ref, o_ref):
    xw = lax.dot_general(
        x_ref[...], w1_ref[...], (((1,), (1,)), ((), ())),
        preferred_element_type=jnp.float32,
    )
    o_ref[...] = xw + b1_ref[...]


def _tc_scale_body(degp_ref, xw_ref, g_ref):
    dis = _dis_from_parts(degp_ref[...])
    g_ref[...] = dis[:, None] * xw_ref[...]


def _tc_b_body(s_ref, degp_ref, w2_ref, b2_ref, o_ref):
    dis = _dis_from_parts(degp_ref[...])
    h = dis[:, None] * (s_ref[0] + s_ref[1])
    h = jnp.maximum(h, 0.0)
    hw = lax.dot_general(
        h, w2_ref[...], (((1,), (1,)), ((), ())),
        preferred_element_type=jnp.float32,
    )
    o_ref[...] = dis[:, None] * (hw + b2_ref[...])


def _tc_c_body(s_ref, degp_ref, o_ref):
    dis = _dis_from_parts(degp_ref[...])
    z = dis[:, None] * (s_ref[0] + s_ref[1])
    col = lax.broadcasted_iota(jnp.int32, z.shape, 1)
    valid = col < _C
    zm = jnp.where(valid, z, -jnp.inf)
    m = jnp.max(zm, axis=1, keepdims=True)
    e = jnp.where(valid, jnp.exp(z - m), 0.0)
    s = jnp.sum(e, axis=1, keepdims=True)
    o_ref[...] = z - m - jnp.log(s)


def _row_spec(shape_prefix=()):
    nd = len(shape_prefix)
    return pl.BlockSpec(
        shape_prefix + (_TC_BLOCK, _H),
        lambda i: (0,) * nd + (i, 0),
    )


_degp_spec = pl.BlockSpec((_NW, _TC_BLOCK), lambda i: (0, i))


def _tc_mm1(x, W1, b1r):
    return pl.pallas_call(
        _tc_mm1_body,
        grid=(_GRID,),
        in_specs=[
            pl.BlockSpec((_TC_BLOCK, _D), lambda i: (i, 0)),
            pl.BlockSpec((_H, _D), lambda i: (0, 0)),
            pl.BlockSpec((1, _H), lambda i: (0, 0)),
        ],
        out_specs=_row_spec(),
        out_shape=jax.ShapeDtypeStruct((_NA, _H), jnp.float32),
    )(x, W1, b1r)


def _tc_scale(degp, xw):
    return pl.pallas_call(
        _tc_scale_body,
        grid=(_GRID,),
        in_specs=[_degp_spec, _row_spec()],
        out_specs=_row_spec(),
        out_shape=jax.ShapeDtypeStruct((_NA, _H), jnp.float32),
    )(degp, xw)


def _tc_b(s1, degp, W2p, b2r):
    return pl.pallas_call(
        _tc_b_body,
        grid=(_GRID,),
        in_specs=[
            _row_spec((2,)),
            _degp_spec,
            pl.BlockSpec((_H, _H), lambda i: (0, 0)),
            pl.BlockSpec((1, _H), lambda i: (0, 0)),
        ],
        out_specs=_row_spec(),
        out_shape=jax.ShapeDtypeStruct((_NA, _H), jnp.float32),
    )(s1, degp, W2p, b2r)


def _tc_c(s2, degp):
    return pl.pallas_call(
        _tc_c_body,
        grid=(_GRID,),
        in_specs=[_row_spec((2,)), _degp_spec],
        out_specs=_row_spec(),
        out_shape=jax.ShapeDtypeStruct((_N, _H), jnp.float32),
    )(s2, degp)


def kernel(x, edge_index, W1, b1, W2, b2):
    ei3 = edge_index.reshape(2, _NROWS, _K)

    b1r = b1.reshape(1, _H)
    W2p = jnp.pad(W2, ((0, _H - _C), (0, 0)))
    b2r = jnp.pad(b2, (0, _H - _C)).reshape(1, _H)
    zero_h = jnp.zeros((_NA, _H), jnp.float32)

    degp = _sc_degree(ei3)
    xw1 = _tc_mm1(x, W1, b1r)
    g1 = _tc_scale(degp, xw1)
    s1 = _sc_scatter(g1, ei3, zero_h)
    g2 = _tc_b(s1, degp, W2p, b2r)
    s2 = _sc_scatter(g2, ei3, zero_h)
    out = _tc_c(s2, degp)
    return out[:, :_C]
